# baseline (speedup 1.0000x reference)
import functools

import jax
import jax.numpy as jnp
from jax import lax
from jax.experimental import pallas as pl
from jax.experimental.pallas import tpu as pltpu
from jax.experimental.pallas import tpu_sc as plsc

_C = 384
_MUL = 1.0 / 40.0
_T_MAX = 2.0
_ROWS = 4096
_NW = 32
_RPW = _ROWS // _NW
_CHUNK = 16
_NCH = _RPW // _CHUNK

_mesh = plsc.VectorSubcoreMesh(core_axis_name="c", subcore_axis_name="s")


@functools.partial(
    pl.kernel,
    out_type=jax.ShapeDtypeStruct((_ROWS, 2 * _C), jnp.float32),
    mesh=_mesh,
    scratch_types=[
        pltpu.VMEM((_CHUNK, 2 * _C), jnp.float32),
        pltpu.VMEM((_CHUNK, 2 * _C), jnp.float32),
        pltpu.VMEM((_CHUNK, 2 * _C), jnp.float32),
    ],
)
def _sc_body(t1_hbm, t2_hbm, out_hbm, b1, b2, bo):
    wid = lax.axis_index("s") * 2 + lax.axis_index("c")
    base = wid * _RPW
    for ch in range(_NCH):
        off = base + ch * _CHUNK
        pltpu.sync_copy(t1_hbm.at[pl.ds(off, _CHUNK)], b1)
        pltpu.sync_copy(t2_hbm.at[pl.ds(off, _CHUNK)], b2)

        def row(r, carry):
            def col(j, carry2):
                s = j * 16
                alo = b1[r, pl.ds(s, 16)]
                ahi = b1[r, pl.ds(s + _C, 16)]
                blo = b2[r, pl.ds(s, 16)]
                bhi = b2[r, pl.ds(s + _C, 16)]
                d = ((alo - ahi) + (blo - bhi)) * _MUL
                bo[r, pl.ds(s, 16)] = jnp.minimum(d + _T_MAX, _T_MAX)
                bo[r, pl.ds(s + _C, 16)] = jnp.minimum(_T_MAX - d, _T_MAX)
                return carry2

            return lax.fori_loop(0, _C // 16, col, carry)

        lax.fori_loop(0, _CHUNK, row, 0)
        pltpu.sync_copy(bo, out_hbm.at[pl.ds(off, _CHUNK)])


def kernel(tj1, tj2):
    t1 = jnp.transpose(tj1, (0, 2, 3, 1)).reshape(_ROWS, 2 * _C)
    t2 = jnp.transpose(tj2, (0, 2, 3, 1)).reshape(_ROWS, 2 * _C)
    out = _sc_body(t1, t2)
    return jnp.transpose(out.reshape(64, 64, 2 * _C), (2, 0, 1))


# speedup vs baseline: 5.8161x; 5.8161x over previous
import jax
import jax.numpy as jnp
from jax.experimental import pallas as pl
from jax.experimental.pallas import tpu as pltpu

_C = 384
_MUL = 1.0 / 40.0
_T_MAX = 2.0
_CH = 2
_N = 64 // _CH
_NBI = 14
_NBO = 7


def _body(t1_ref, t2_ref, out_ref, ibuf, obuf, isem, osem):
    def in_copy(slot, i, k):
        src = (t1_ref, t2_ref)[k]
        return pltpu.make_async_copy(
            src.at[0, pl.ds(i * _CH, _CH)], ibuf.at[slot, k],
            isem.at[slot, k])

    def out_copy(slot, i):
        return pltpu.make_async_copy(
            obuf.at[slot], out_ref.at[pl.ds(i * _CH, _CH)], osem.at[slot])

    def start_in(slot, i):
        in_copy(slot, i, 0).start()
        in_copy(slot, i, 1).start()

    for b in range(min(_NBI, _N)):
        start_in(b, b)

    def step(i, carry):
        si = jax.lax.rem(i, _NBI)
        so = jax.lax.rem(i, _NBO)
        in_copy(si, i, 0).wait()
        in_copy(si, i, 1).wait()

        @pl.when(i >= _NBO)
        def _():
            out_copy(so, i - _NBO).wait()

        a = ibuf[si, 0]
        b = ibuf[si, 1]
        d = ((a[..., :_C] - a[..., _C:]) + (b[..., :_C] - b[..., _C:])) * _MUL
        obuf[so, :, :, :_C] = jnp.minimum(d + _T_MAX, _T_MAX)
        obuf[so, :, :, _C:] = jnp.minimum(_T_MAX - d, _T_MAX)
        out_copy(so, i).start()

        @pl.when(i + _NBI < _N)
        def _():
            start_in(si, i + _NBI)
        return carry

    jax.lax.fori_loop(0, _N, step, 0)

    for b in range(min(_NBO, _N)):
        i = _N - min(_NBO, _N) + b
        out_copy(i % _NBO, i).wait()


def kernel(tj1, tj2):
    t1 = jnp.transpose(tj1, (0, 2, 3, 1))
    t2 = jnp.transpose(tj2, (0, 2, 3, 1))
    out = pl.pallas_call(
        _body,
        in_specs=[pl.BlockSpec(memory_space=pl.ANY),
                  pl.BlockSpec(memory_space=pl.ANY)],
        out_specs=pl.BlockSpec(memory_space=pl.ANY),
        out_shape=jax.ShapeDtypeStruct((64, 64, 2 * _C), jnp.float32),
        scratch_shapes=[
            pltpu.VMEM((_NBI, 2, _CH, 64, 2 * _C), jnp.float32),
            pltpu.VMEM((_NBO, _CH, 64, 2 * _C), jnp.float32),
            pltpu.SemaphoreType.DMA((_NBI, 2)),
            pltpu.SemaphoreType.DMA((_NBO,)),
        ],
    )(t1, t2)
    return jnp.transpose(out, (2, 0, 1))
